# baseline (speedup 1.0000x reference)
import jax
import jax.numpy as jnp
from jax import lax
from jax.experimental import pallas as pl
from jax.experimental.pallas import tpu as pltpu
from jax.experimental.pallas import tpu_sc as plsc

F = 26
D = 64
B = 4096
SEG = 38462
TOTAL = F * SEG
NC, NS = 2, 16
NW = NC * NS
DQ = 8
NTASK = F * (D // DQ)
WWIN = 4096
NWIN = 10
TAIL0 = (TOTAL // 128) * 128
WMAX = ((TOTAL - WWIN) // 128) * 128


def _sc_body(tableT, sb_hbm, sv_hbm, bnd_hbm, tail, outT,
             sb_v, sv_v, bnd_v, win_v, acc_v, tail_v, sem0, sem1):
    cc = lax.axis_index("c")
    ss = lax.axis_index("s")
    wid = ss * NC + cc
    iota = lax.broadcasted_iota(jnp.int32, (16,), 0)
    pltpu.sync_copy(tail, tail_v)
    sems = (sem0, sem1)

    def taskloop(ti, tcarry):
        t = ti * NW + wid

        @pl.when(t < NTASK)
        def _task():
            f = t % F
            dq = t // F
            d0 = pl.multiple_of(dq * DQ, DQ)
            fb = pl.multiple_of(f * B, B)
            pltpu.sync_copy(sb_hbm.at[pl.ds(fb, B)], sb_v.at[pl.ds(0, B)])
            pltpu.sync_copy(sv_hbm.at[pl.ds(fb, B)], sv_v.at[pl.ds(0, B)])
            pltpu.sync_copy(bnd_hbm.at[pl.ds(pl.multiple_of(f * 128, 128),
                                             128)], bnd_v)
            w0s = pl.multiple_of(f * SEG // 128 * 128, 128)

            def w0_of(wi):
                return pl.multiple_of(
                    jnp.minimum(w0s + wi * WWIN, WMAX), 128)

            def issue(wi, buf, sem):
                pltpu.async_copy(
                    tableT.at[pl.ds(d0, DQ), pl.ds(w0_of(wi), WWIN)],
                    win_v.at[buf], sem)

            def wait(wi, buf, sem):
                pltpu.make_async_copy(
                    tableT.at[pl.ds(d0, DQ), pl.ds(w0_of(wi), WWIN)],
                    win_v.at[buf], sem).wait()

            def process(wi, buf):
                w0 = w0_of(wi)
                s0 = bnd_v[pl.ds(wi, 16)][0]
                e0 = bnd_v[pl.ds(64 + wi, 16)][0]
                wv = win_v.at[buf]

                def hblock(bi, _):
                    h0 = s0 + bi * 16
                    hb = sb_v[pl.ds(h0, 16)] & (B - 1)
                    hv = sv_v[pl.ds(h0, 16)]
                    vm = (h0 + iota) < e0
                    cvec = (hv - w0) & (WWIN - 1)
                    for dl in range(DQ):
                        dsp = jnp.full((16,), dl, jnp.int32)
                        vals = plsc.load_gather(wv, [dsp, cvec])
                        plsc.store_scatter(acc_v, [dsp, hb], vals, mask=vm)
                    return _

                lax.fori_loop(0, (e0 - s0 + 15) >> 4, hblock, 0)

            issue(0, 0, sem0)
            issue(1, 1, sem1)

            def wpair(p, carry):
                w_a = 2 * p
                wait(w_a, 0, sem0)
                process(w_a, 0)

                @pl.when(w_a + 2 < NWIN)
                def _():
                    issue(w_a + 2, 0, sem0)

                wait(w_a + 1, 1, sem1)
                process(w_a + 1, 1)

                @pl.when(w_a + 3 < NWIN)
                def _():
                    issue(w_a + 3, 1, sem1)
                return carry

            lax.fori_loop(0, NWIN // 2, wpair, 0)

            s0 = bnd_v[pl.ds(NWIN, 16)][0]
            e0 = bnd_v[pl.ds(64 + NWIN, 16)][0]

            def tblock(bi, _):
                h0 = s0 + bi * 16
                hb = sb_v[pl.ds(h0, 16)] & (B - 1)
                hv = sv_v[pl.ds(h0, 16)]
                vm = (h0 + iota) < e0
                rvec = (hv - TAIL0) & 127
                for dl in range(DQ):
                    dsp = jnp.full((16,), dl, jnp.int32)
                    vals = plsc.load_gather(tail_v, [rvec, d0 + dsp])
                    plsc.store_scatter(acc_v, [dsp, hb], vals, mask=vm)
                return _

            lax.fori_loop(0, (e0 - s0 + 15) >> 4, tblock, 0)

            pltpu.sync_copy(
                acc_v,
                outT.at[pl.ds(pl.multiple_of(f * D + dq * DQ, DQ), DQ)])
        return tcarry

    lax.fori_loop(0, (NTASK + NW - 1) // NW, taskloop, 0)


def _sc_gather_t(tableT, sb, sv, bnd, tail):
    mesh = plsc.VectorSubcoreMesh(
        core_axis_name="c", subcore_axis_name="s", num_cores=NC, num_subcores=NS
    )
    return pl.kernel(
        _sc_body,
        out_type=jax.ShapeDtypeStruct((F * D, B), jnp.float32),
        mesh=mesh,
        scratch_types=[
            pltpu.VMEM((B + 16,), jnp.int32),
            pltpu.VMEM((B + 16,), jnp.int32),
            pltpu.VMEM((128,), jnp.int32),
            pltpu.VMEM((2, DQ, WWIN), jnp.float32),
            pltpu.VMEM((DQ, B), jnp.float32),
            pltpu.VMEM((128, 128), jnp.float32),
            pltpu.SemaphoreType.DMA,
            pltpu.SemaphoreType.DMA,
        ],
        compiler_params=pltpu.CompilerParams(
            use_tc_tiling_on_sc=True, needs_layout_passes=False),
        name="sc_emb_gather_t",
    )(tableT, sb, sv, bnd, tail)


def _mlp_body(x_ref, w1, b1, g1, be1, rm1, rv1, w2, b2, g2, be2, rm2, rv2,
              w3, b3, g3, be3, rm3, rv3, wo, bo, out_ref):
    h = lax.dot_general(x_ref[...], w1[...], (((0,), (1,)), ((), ())),
                        preferred_element_type=jnp.float32)
    for first, (w, b, g, be, rm, rv) in (
            (True, (w1, b1, g1, be1, rm1, rv1)),
            (False, (w2, b2, g2, be2, rm2, rv2)),
            (False, (w3, b3, g3, be3, rm3, rv3))):
        if not first:
            h = lax.dot_general(h, w[...], (((1,), (1,)), ((), ())),
                                preferred_element_type=jnp.float32)
        scale = g[...] * lax.rsqrt(rv[...] + 1e-5)
        h = (h + (b[...] - rm[...])) * scale + be[...]
        h = jnp.maximum(h, 0.0)
    o = lax.dot_general(wo[...], h, (((1,), (1,)), ((), ())),
                        preferred_element_type=jnp.float32)
    out_ref[...] = jax.nn.sigmoid(o + bo[...])[0]


def _tc_mlp(xT, w1, b1, g1, be1, rm1, rv1, w2, b2, g2, be2, rm2, rv2,
            w3, b3, g3, be3, rm3, rv3, wo, bo):
    blk = 512
    grid = (B // blk,)
    full = lambda shape: pl.BlockSpec(shape, lambda m: (0,) * len(shape))
    in_specs = [
        pl.BlockSpec((F * D, blk), lambda m: (0, m)),
        full((1024, F * D)), full((1024,)), full((1024,)), full((1024,)),
        full((1024,)), full((1024,)),
        full((512, 1024)), full((512,)), full((512,)), full((512,)),
        full((512,)), full((512,)),
        full((256, 512)), full((256,)), full((256,)), full((256,)),
        full((256,)), full((256,)),
        full((1, 256)), full((1, 1)),
    ]
    out = pl.pallas_call(
        _mlp_body,
        grid=grid,
        in_specs=in_specs,
        out_specs=pl.BlockSpec((blk,), lambda m: (m,)),
        out_shape=jax.ShapeDtypeStruct((B,), jnp.float32),
        name="tc_mlp",
    )(xT, w1, b1, g1, be1, rm1, rv1, w2, b2, g2, be2, rm2, rv2,
      w3, b3, g3, be3, rm3, rv3, wo, bo.reshape(1, 1))
    return out


def kernel(field, emb_table, W1, b1, g1, be1, rm1, rv1, W2, b2, g2, be2,
           rm2, rv2, W3, b3, g3, be3, rm3, rv3, Wo, bo):
    offsets = jnp.arange(F, dtype=jnp.int32) * SEG
    fieldT = field.T
    bio = lax.broadcasted_iota(jnp.int32, (F, B), 1)
    w0s = (offsets // 128) * 128
    idxT = fieldT + offsets[:, None]
    shift = WWIN.bit_length() - 1
    win = jnp.where(idxT >= TAIL0, NWIN, (idxT - w0s[:, None]) >> shift)
    oh = (win[:, :, None] ==
          jnp.arange(NWIN + 1)[None, None, :]).astype(jnp.int32)
    counts = jnp.sum(oh, axis=1, dtype=jnp.int32)
    csum = jnp.cumsum(counts, axis=1, dtype=jnp.int32)
    starts = csum - counts
    ends = csum
    rank = jnp.sum(jnp.cumsum(oh, axis=1) * oh, axis=2) - 1
    pos = jnp.take_along_axis(starts, win, axis=1) + rank
    rowi = lax.broadcasted_iota(jnp.int32, (F, B), 0)
    sbT = jnp.zeros((F, B), jnp.int32).at[rowi, pos].set(
        bio, unique_indices=True)
    svT = jnp.zeros((F, B), jnp.int32).at[rowi, pos].set(
        idxT, unique_indices=True)
    bnd = jnp.zeros((F, 128), jnp.int32)
    bnd = bnd.at[:, :NWIN + 1].set(starts).at[:, 64:64 + NWIN + 1].set(ends)
    tableT = emb_table.T
    tail = jnp.pad(emb_table[TAIL0:, :],
                   ((0, 128 - (TOTAL - TAIL0)), (0, 64)))
    xT = _sc_gather_t(tableT, sbT.reshape(-1), svT.reshape(-1),
                      bnd.reshape(-1), tail)
    W1p = W1.reshape(1024, D, F).transpose(0, 2, 1).reshape(1024, F * D)
    return _tc_mlp(xT, W1p, b1, g1, be1, rm1, rv1, W2, b2, g2, be2, rm2, rv2,
                   W3, b3, g3, be3, rm3, rv3, Wo, bo)


# speedup vs baseline: 7.4800x; 7.4800x over previous
import jax
import jax.numpy as jnp
from jax import lax
from jax.experimental import pallas as pl
from jax.experimental.pallas import tpu as pltpu
from jax.experimental.pallas import tpu_sc as plsc

F = 26
D = 64
B = 4096
SEG = 38462
TOTAL = F * SEG
NC, NS = 2, 16
NW = NC * NS
DQ = 8
NTASK = F * (D // DQ)
WWIN = 4096
NWIN = 10
TAIL0 = (TOTAL // 128) * 128
WMAX = ((TOTAL - WWIN) // 128) * 128


def _sc_body(tableT, sb_hbm, sv_hbm, bnd_hbm, tail, outT,
             sb_v, sv_v, bnd_v, win_v, acc_v, tail_v, sem0, sem1):
    cc = lax.axis_index("c")
    ss = lax.axis_index("s")
    wid = ss * NC + cc
    iota = lax.broadcasted_iota(jnp.int32, (16,), 0)
    pltpu.sync_copy(tail, tail_v)
    sems = (sem0, sem1)

    def taskloop(ti, tcarry):
        t = ti * NW + wid

        @pl.when(t < NTASK)
        def _task():
            f = t % F
            dq = t // F
            d0 = pl.multiple_of(dq * DQ, DQ)
            fb = pl.multiple_of(f * B, B)
            pltpu.sync_copy(sb_hbm.at[pl.ds(fb, B)], sb_v.at[pl.ds(0, B)])
            pltpu.sync_copy(sv_hbm.at[pl.ds(fb, B)], sv_v.at[pl.ds(0, B)])
            pltpu.sync_copy(bnd_hbm.at[pl.ds(pl.multiple_of(f * 128, 128),
                                             128)], bnd_v)
            w0s = pl.multiple_of(f * SEG // 128 * 128, 128)

            def w0_of(wi):
                return pl.multiple_of(
                    jnp.minimum(w0s + wi * WWIN, WMAX), 128)

            def issue(wi, buf, sem):
                pltpu.async_copy(
                    tableT.at[pl.ds(d0, DQ), pl.ds(w0_of(wi), WWIN)],
                    win_v.at[buf], sem)

            def wait(wi, buf, sem):
                pltpu.make_async_copy(
                    tableT.at[pl.ds(d0, DQ), pl.ds(w0_of(wi), WWIN)],
                    win_v.at[buf], sem).wait()

            def process(wi, buf):
                w0 = w0_of(wi)
                s0 = bnd_v[pl.ds(wi, 16)][0]
                e0 = bnd_v[pl.ds(64 + wi, 16)][0]
                wv = win_v.at[buf]

                def hblock(bi, _):
                    h0 = s0 + bi * 16
                    hb = sb_v[pl.ds(h0, 16)] & (B - 1)
                    hv = sv_v[pl.ds(h0, 16)]
                    vm = (h0 + iota) < e0
                    cvec = (hv - w0) & (WWIN - 1)
                    for dl in range(DQ):
                        dsp = jnp.full((16,), dl, jnp.int32)
                        vals = plsc.load_gather(wv, [dsp, cvec])
                        plsc.store_scatter(acc_v, [dsp, hb], vals, mask=vm)
                    return _

                lax.fori_loop(0, (e0 - s0 + 15) >> 4, hblock, 0)

            issue(0, 0, sem0)
            issue(1, 1, sem1)

            def wpair(p, carry):
                w_a = 2 * p
                wait(w_a, 0, sem0)
                process(w_a, 0)

                @pl.when(w_a + 2 < NWIN)
                def _():
                    issue(w_a + 2, 0, sem0)

                wait(w_a + 1, 1, sem1)
                process(w_a + 1, 1)

                @pl.when(w_a + 3 < NWIN)
                def _():
                    issue(w_a + 3, 1, sem1)
                return carry

            lax.fori_loop(0, NWIN // 2, wpair, 0)

            s0 = bnd_v[pl.ds(NWIN, 16)][0]
            e0 = bnd_v[pl.ds(64 + NWIN, 16)][0]

            def tblock(bi, _):
                h0 = s0 + bi * 16
                hb = sb_v[pl.ds(h0, 16)] & (B - 1)
                hv = sv_v[pl.ds(h0, 16)]
                vm = (h0 + iota) < e0
                rvec = (hv - TAIL0) & 127
                for dl in range(DQ):
                    dsp = jnp.full((16,), dl, jnp.int32)
                    vals = plsc.load_gather(tail_v, [rvec, d0 + dsp])
                    plsc.store_scatter(acc_v, [dsp, hb], vals, mask=vm)
                return _

            lax.fori_loop(0, (e0 - s0 + 15) >> 4, tblock, 0)

            pltpu.sync_copy(
                acc_v,
                outT.at[pl.ds(pl.multiple_of(f * D + dq * DQ, DQ), DQ)])
        return tcarry

    lax.fori_loop(0, (NTASK + NW - 1) // NW, taskloop, 0)


def _sc_gather_t(tableT, sb, sv, bnd, tail):
    mesh = plsc.VectorSubcoreMesh(
        core_axis_name="c", subcore_axis_name="s", num_cores=NC, num_subcores=NS
    )
    return pl.kernel(
        _sc_body,
        out_type=jax.ShapeDtypeStruct((F * D, B), jnp.float32),
        mesh=mesh,
        scratch_types=[
            pltpu.VMEM((B + 16,), jnp.int32),
            pltpu.VMEM((B + 16,), jnp.int32),
            pltpu.VMEM((128,), jnp.int32),
            pltpu.VMEM((2, DQ, WWIN), jnp.float32),
            pltpu.VMEM((DQ, B), jnp.float32),
            pltpu.VMEM((128, 128), jnp.float32),
            pltpu.SemaphoreType.DMA,
            pltpu.SemaphoreType.DMA,
        ],
        compiler_params=pltpu.CompilerParams(
            use_tc_tiling_on_sc=True, needs_layout_passes=False),
        name="sc_emb_gather_t",
    )(tableT, sb, sv, bnd, tail)


def _mlp_body(x_ref, w1, b1, g1, be1, rm1, rv1, w2, b2, g2, be2, rm2, rv2,
              w3, b3, g3, be3, rm3, rv3, wo, bo, out_ref):
    h = lax.dot_general(x_ref[...], w1[...], (((0,), (1,)), ((), ())),
                        preferred_element_type=jnp.float32)
    for first, (w, b, g, be, rm, rv) in (
            (True, (w1, b1, g1, be1, rm1, rv1)),
            (False, (w2, b2, g2, be2, rm2, rv2)),
            (False, (w3, b3, g3, be3, rm3, rv3))):
        if not first:
            h = lax.dot_general(h, w[...], (((1,), (1,)), ((), ())),
                                preferred_element_type=jnp.float32)
        scale = g[...] * lax.rsqrt(rv[...] + 1e-5)
        h = (h + (b[...] - rm[...])) * scale + be[...]
        h = jnp.maximum(h, 0.0)
    o = lax.dot_general(wo[...], h, (((1,), (1,)), ((), ())),
                        preferred_element_type=jnp.float32)
    out_ref[...] = jax.nn.sigmoid(o + bo[...])[0]


def _tc_mlp(xT, w1, b1, g1, be1, rm1, rv1, w2, b2, g2, be2, rm2, rv2,
            w3, b3, g3, be3, rm3, rv3, wo, bo):
    blk = 512
    grid = (B // blk,)
    full = lambda shape: pl.BlockSpec(shape, lambda m: (0,) * len(shape))
    in_specs = [
        pl.BlockSpec((F * D, blk), lambda m: (0, m)),
        full((1024, F * D)), full((1024,)), full((1024,)), full((1024,)),
        full((1024,)), full((1024,)),
        full((512, 1024)), full((512,)), full((512,)), full((512,)),
        full((512,)), full((512,)),
        full((256, 512)), full((256,)), full((256,)), full((256,)),
        full((256,)), full((256,)),
        full((1, 256)), full((1, 1)),
    ]
    out = pl.pallas_call(
        _mlp_body,
        grid=grid,
        in_specs=in_specs,
        out_specs=pl.BlockSpec((blk,), lambda m: (m,)),
        out_shape=jax.ShapeDtypeStruct((B,), jnp.float32),
        name="tc_mlp",
    )(xT, w1, b1, g1, be1, rm1, rv1, w2, b2, g2, be2, rm2, rv2,
      w3, b3, g3, be3, rm3, rv3, wo, bo.reshape(1, 1))
    return out


def kernel(field, emb_table, W1, b1, g1, be1, rm1, rv1, W2, b2, g2, be2,
           rm2, rv2, W3, b3, g3, be3, rm3, rv3, Wo, bo):
    offsets = jnp.arange(F, dtype=jnp.int32) * SEG
    fieldT = field.T
    bio = lax.broadcasted_iota(jnp.int32, (F, B), 1)
    w0s = (offsets // 128) * 128
    idxT = fieldT + offsets[:, None]
    shift = WWIN.bit_length() - 1
    win = jnp.where(idxT >= TAIL0, NWIN, (idxT - w0s[:, None]) >> shift)
    counts = jnp.sum(
        win[:, :, None] == jnp.arange(NWIN + 1)[None, None, :], axis=1,
        dtype=jnp.int32)
    csum = jnp.cumsum(counts, axis=1, dtype=jnp.int32)
    starts = csum - counts
    ends = csum
    sp = jnp.sort(fieldT * B + bio, axis=1)
    sbT = sp & (B - 1)
    svT = (sp >> 12) + offsets[:, None]
    bnd = jnp.zeros((F, 128), jnp.int32)
    bnd = bnd.at[:, :NWIN + 1].set(starts).at[:, 64:64 + NWIN + 1].set(ends)
    tableT = emb_table.T
    tail = jnp.pad(emb_table[TAIL0:, :],
                   ((0, 128 - (TOTAL - TAIL0)), (0, 64)))
    xT = _sc_gather_t(tableT, sbT.reshape(-1), svT.reshape(-1),
                      bnd.reshape(-1), tail)
    W1p = W1.reshape(1024, D, F).transpose(0, 2, 1).reshape(1024, F * D)
    return _tc_mlp(xT, W1p, b1, g1, be1, rm1, rv1, W2, b2, g2, be2, rm2, rv2,
                   W3, b3, g3, be3, rm3, rv3, Wo, bo)


# speedup vs baseline: 7.7663x; 1.0383x over previous
import jax
import jax.numpy as jnp
from jax import lax
from jax.experimental import pallas as pl
from jax.experimental.pallas import tpu as pltpu
from jax.experimental.pallas import tpu_sc as plsc

F = 26
D = 64
B = 4096
SEG = 38462
TOTAL = F * SEG
NC, NS = 2, 16
NW = NC * NS
DQ = 8
NTASK = F * (D // DQ)
WWIN = 4096
NWIN = 10
TAIL0 = (TOTAL // 128) * 128
WMAX = ((TOTAL - WWIN) // 128) * 128


def _sc_body(tableT, sb_hbm, sv_hbm, bnd_hbm, tail, outT,
             sb_v, sv_v, bnd_v, win_v, acc_v, tail_v, sem0, sem1):
    cc = lax.axis_index("c")
    ss = lax.axis_index("s")
    wid = ss * NC + cc
    iota = lax.broadcasted_iota(jnp.int32, (16,), 0)
    pltpu.sync_copy(tail, tail_v)
    sems = (sem0, sem1)

    def taskloop(ti, tcarry):
        t = ti * NW + wid

        @pl.when(t < NTASK)
        def _task():
            f = t % F
            dq = t // F
            d0 = pl.multiple_of(dq * DQ, DQ)
            fb = pl.multiple_of(f * B, B)
            w0s = pl.multiple_of(f * SEG // 128 * 128, 128)

            def w0_of(wi):
                return pl.multiple_of(
                    jnp.minimum(w0s + wi * WWIN, WMAX), 128)

            def issue(wi, buf, sem):
                pltpu.async_copy(
                    tableT.at[pl.ds(d0, DQ), pl.ds(w0_of(wi), WWIN)],
                    win_v.at[buf], sem)

            def wait(wi, buf, sem):
                pltpu.make_async_copy(
                    tableT.at[pl.ds(d0, DQ), pl.ds(w0_of(wi), WWIN)],
                    win_v.at[buf], sem).wait()

            def process(wi, buf):
                w0 = w0_of(wi)
                s0 = bnd_v[pl.ds(wi, 16)][0]
                e0 = bnd_v[pl.ds(64 + wi, 16)][0]
                wv = win_v.at[buf]

                def hblock(bi, _):
                    h0 = s0 + bi * 16
                    hb = sb_v[pl.ds(h0, 16)] & (B - 1)
                    hv = sv_v[pl.ds(h0, 16)]
                    vm = (h0 + iota) < e0
                    cvec = (hv - w0) & (WWIN - 1)
                    for dl in range(DQ):
                        dsp = jnp.full((16,), dl, jnp.int32)
                        vals = plsc.load_gather(wv, [dsp, cvec])
                        plsc.store_scatter(acc_v, [dsp, hb], vals, mask=vm)
                    return _

                lax.fori_loop(0, (e0 - s0 + 15) >> 4, hblock, 0)

            issue(0, 0, sem0)
            issue(1, 1, sem1)
            pltpu.sync_copy(sb_hbm.at[pl.ds(fb, B)], sb_v.at[pl.ds(0, B)])
            pltpu.sync_copy(sv_hbm.at[pl.ds(fb, B)], sv_v.at[pl.ds(0, B)])
            pltpu.sync_copy(bnd_hbm.at[pl.ds(pl.multiple_of(f * 128, 128),
                                             128)], bnd_v)

            def wpair(p, carry):
                w_a = 2 * p
                wait(w_a, 0, sem0)
                process(w_a, 0)

                @pl.when(w_a + 2 < NWIN)
                def _():
                    issue(w_a + 2, 0, sem0)

                wait(w_a + 1, 1, sem1)
                process(w_a + 1, 1)

                @pl.when(w_a + 3 < NWIN)
                def _():
                    issue(w_a + 3, 1, sem1)
                return carry

            lax.fori_loop(0, NWIN // 2, wpair, 0)

            s0 = bnd_v[pl.ds(NWIN, 16)][0]
            e0 = bnd_v[pl.ds(64 + NWIN, 16)][0]

            def tblock(bi, _):
                h0 = s0 + bi * 16
                hb = sb_v[pl.ds(h0, 16)] & (B - 1)
                hv = sv_v[pl.ds(h0, 16)]
                vm = (h0 + iota) < e0
                rvec = (hv - TAIL0) & 127
                for dl in range(DQ):
                    dsp = jnp.full((16,), dl, jnp.int32)
                    vals = plsc.load_gather(tail_v, [rvec, d0 + dsp])
                    plsc.store_scatter(acc_v, [dsp, hb], vals, mask=vm)
                return _

            lax.fori_loop(0, (e0 - s0 + 15) >> 4, tblock, 0)

            pltpu.sync_copy(
                acc_v,
                outT.at[pl.ds(pl.multiple_of(f * D + dq * DQ, DQ), DQ)])
        return tcarry

    lax.fori_loop(0, (NTASK + NW - 1) // NW, taskloop, 0)


def _sc_gather_t(tableT, sb, sv, bnd, tail):
    mesh = plsc.VectorSubcoreMesh(
        core_axis_name="c", subcore_axis_name="s", num_cores=NC, num_subcores=NS
    )
    return pl.kernel(
        _sc_body,
        out_type=jax.ShapeDtypeStruct((F * D, B), jnp.float32),
        mesh=mesh,
        scratch_types=[
            pltpu.VMEM((B + 16,), jnp.int32),
            pltpu.VMEM((B + 16,), jnp.int32),
            pltpu.VMEM((128,), jnp.int32),
            pltpu.VMEM((2, DQ, WWIN), jnp.float32),
            pltpu.VMEM((DQ, B), jnp.float32),
            pltpu.VMEM((128, 128), jnp.float32),
            pltpu.SemaphoreType.DMA,
            pltpu.SemaphoreType.DMA,
        ],
        compiler_params=pltpu.CompilerParams(
            use_tc_tiling_on_sc=True, needs_layout_passes=False),
        name="sc_emb_gather_t",
    )(tableT, sb, sv, bnd, tail)


def _mlp_body(x_ref, w1, b1, g1, be1, rm1, rv1, w2, b2, g2, be2, rm2, rv2,
              w3, b3, g3, be3, rm3, rv3, wo, bo, out_ref):
    h = lax.dot_general(x_ref[...], w1[...], (((0,), (1,)), ((), ())),
                        preferred_element_type=jnp.float32)
    for first, (w, b, g, be, rm, rv) in (
            (True, (w1, b1, g1, be1, rm1, rv1)),
            (False, (w2, b2, g2, be2, rm2, rv2)),
            (False, (w3, b3, g3, be3, rm3, rv3))):
        if not first:
            h = lax.dot_general(h, w[...], (((1,), (1,)), ((), ())),
                                preferred_element_type=jnp.float32)
        scale = g[...] * lax.rsqrt(rv[...] + 1e-5)
        h = (h + (b[...] - rm[...])) * scale + be[...]
        h = jnp.maximum(h, 0.0)
    o = lax.dot_general(wo[...], h, (((1,), (1,)), ((), ())),
                        preferred_element_type=jnp.float32)
    out_ref[...] = jax.nn.sigmoid(o + bo[...])[0]


def _tc_mlp(xT, w1, b1, g1, be1, rm1, rv1, w2, b2, g2, be2, rm2, rv2,
            w3, b3, g3, be3, rm3, rv3, wo, bo):
    blk = 1024
    grid = (B // blk,)
    full = lambda shape: pl.BlockSpec(shape, lambda m: (0,) * len(shape))
    in_specs = [
        pl.BlockSpec((F * D, blk), lambda m: (0, m)),
        full((1024, F * D)), full((1024,)), full((1024,)), full((1024,)),
        full((1024,)), full((1024,)),
        full((512, 1024)), full((512,)), full((512,)), full((512,)),
        full((512,)), full((512,)),
        full((256, 512)), full((256,)), full((256,)), full((256,)),
        full((256,)), full((256,)),
        full((1, 256)), full((1, 1)),
    ]
    out = pl.pallas_call(
        _mlp_body,
        grid=grid,
        in_specs=in_specs,
        out_specs=pl.BlockSpec((blk,), lambda m: (m,)),
        out_shape=jax.ShapeDtypeStruct((B,), jnp.float32),
        name="tc_mlp",
    )(xT, w1, b1, g1, be1, rm1, rv1, w2, b2, g2, be2, rm2, rv2,
      w3, b3, g3, be3, rm3, rv3, wo, bo.reshape(1, 1))
    return out


def kernel(field, emb_table, W1, b1, g1, be1, rm1, rv1, W2, b2, g2, be2,
           rm2, rv2, W3, b3, g3, be3, rm3, rv3, Wo, bo):
    offsets = jnp.arange(F, dtype=jnp.int32) * SEG
    fieldT = field.T
    bio = lax.broadcasted_iota(jnp.int32, (F, B), 1)
    w0s = (offsets // 128) * 128
    idxT = fieldT + offsets[:, None]
    shift = WWIN.bit_length() - 1
    win = jnp.where(idxT >= TAIL0, NWIN, (idxT - w0s[:, None]) >> shift)
    counts = jnp.sum(
        win[:, :, None] == jnp.arange(NWIN + 1)[None, None, :], axis=1,
        dtype=jnp.int32)
    csum = jnp.cumsum(counts, axis=1, dtype=jnp.int32)
    starts = csum - counts
    ends = csum
    sp = jnp.sort(fieldT * B + bio, axis=1)
    sbT = sp & (B - 1)
    svT = (sp >> 12) + offsets[:, None]
    bnd = jnp.zeros((F, 128), jnp.int32)
    bnd = bnd.at[:, :NWIN + 1].set(starts).at[:, 64:64 + NWIN + 1].set(ends)
    tableT = emb_table.T
    tail = jnp.pad(emb_table[TAIL0:, :],
                   ((0, 128 - (TOTAL - TAIL0)), (0, 64)))
    xT = _sc_gather_t(tableT, sbT.reshape(-1), svT.reshape(-1),
                      bnd.reshape(-1), tail)
    W1p = W1.reshape(1024, D, F).transpose(0, 2, 1).reshape(1024, F * D)
    return _tc_mlp(xT, W1p, b1, g1, be1, rm1, rv1, W2, b2, g2, be2, rm2, rv2,
                   W3, b3, g3, be3, rm3, rv3, Wo, bo)
